# Initial kernel scaffold; baseline (speedup 1.0000x reference)
import jax
import jax.numpy as jnp
from jax import lax
from jax.experimental import pallas as pl
from jax.experimental.pallas import tpu as pltpu
from jax.experimental.pallas import tpu_sc as plsc

_N = 100000
_D = 16
_E = 3200000
_NPAD = 102400
_RT = _NPAD // 16
_ZR = 1600
_NW = 32
_EPAD = 3211264
_ER = _EPAD // 128
_R = 2048


def _sc_pass(with_cnt):
    gk = 4 if with_cnt else 8
    ng = _EPAD // (_NW * gk * 128)
    group = gk * 128

    mesh = plsc.VectorSubcoreMesh(core_axis_name="c", subcore_axis_name="s")
    out_type = [jax.ShapeDtypeStruct((2, _NPAD, _D), jnp.float32)]
    scratch = [
        pltpu.VMEM((gk, 128), jnp.int32),
        pltpu.VMEM((gk, 128), jnp.int32),
        pltpu.VMEM((group, _D), jnp.float32),
        pltpu.VMEM((_ZR, _D), jnp.float32),
        pltpu.VMEM_SHARED((_NPAD, _D), jnp.float32),
        pltpu.SemaphoreType.DMA,
    ]
    if with_cnt:
        out_type.append(jax.ShapeDtypeStruct((2, _NPAD), jnp.float32))
        scratch += [
            pltpu.VMEM((128,), jnp.float32),
            pltpu.VMEM((_RT,), jnp.float32),
            pltpu.VMEM_SHARED((_NPAD,), jnp.float32),
        ]

    def body(h_hbm, src_hbm, dst_hbm, out_hbm, *rest):
        if with_cnt:
            (cnt_hbm, srcb, dstb, rows, zbuf, acc, sem,
             ones, zcnt, cntacc) = rest
        else:
            srcb, dstb, rows, zbuf, acc, sem = rest
        cid = lax.axis_index("c")
        sid = lax.axis_index("s")
        w = cid * 16 + sid

        def _z2(i, carry):
            zbuf[i, :] = jnp.zeros((_D,), jnp.float32)
            return carry
        lax.fori_loop(0, _ZR, _z2, 0)
        for k in range(_RT // _ZR):
            pltpu.sync_copy(zbuf, acc.at[pl.ds(sid * _RT + k * _ZR, _ZR)])
        if with_cnt:
            def _z1(i, carry):
                zcnt[pl.ds(i * 16, 16)] = jnp.zeros((16,), jnp.float32)
                return carry
            lax.fori_loop(0, _RT // 16, _z1, 0)
            pltpu.sync_copy(zcnt, cntacc.at[pl.ds(sid * _RT, _RT)])
            def _o1(i, carry):
                ones[pl.ds(i * 16, 16)] = jnp.full((16,), 1.0, jnp.float32)
                return carry
            lax.fori_loop(0, 8, _o1, 0)
        plsc.subcore_barrier()

        rowbase = w * ng * gk

        def _group(g, carry):
            roff = rowbase + g * gk
            pltpu.sync_copy(src_hbm.at[pl.ds(roff, gk)], srcb)
            pltpu.sync_copy(dst_hbm.at[pl.ds(roff, gk)], dstb)
            cps = [
                pltpu.async_copy(h_hbm.at[srcb.at[j]],
                                 rows.at[pl.ds(j * 128, 128)], sem)
                for j in range(gk)
            ]
            for cp in cps:
                cp.wait()
            for j in range(gk):
                pltpu.sync_copy(rows.at[pl.ds(j * 128, 128)],
                                acc.at[dstb.at[j]], add=True)
                if with_cnt:
                    pltpu.sync_copy(ones, cntacc.at[dstb.at[j]], add=True)
            return carry
        lax.fori_loop(0, ng, _group, 0)
        plsc.subcore_barrier()

        pltpu.sync_copy(acc.at[pl.ds(sid * _RT, _RT)],
                        out_hbm.at[cid, pl.ds(sid * _RT, _RT)])
        if with_cnt:
            pltpu.sync_copy(cntacc.at[pl.ds(sid * _RT, _RT)],
                            cnt_hbm.at[cid, pl.ds(sid * _RT, _RT)])

    return pl.kernel(body, mesh=mesh, out_type=out_type,
                     scratch_types=scratch)


_sc_pass_cnt = _sc_pass(True)
_sc_pass_acc = _sc_pass(False)


def _sage_block(acc_ref, cnt_ref, h_ref, wl_ref, bl_ref, wr_ref):
    a = acc_ref[0] + acc_ref[1]
    c = cnt_ref[:, 0:1] + cnt_ref[:, 1:2]
    mean = a / jnp.maximum(c, 1.0)
    dn = (((1,), (1,)), ((), ()))
    o = (lax.dot_general(mean, wl_ref[...], dn,
                         preferred_element_type=jnp.float32)
         + bl_ref[...]
         + lax.dot_general(h_ref[...], wr_ref[...], dn,
                           preferred_element_type=jnp.float32))
    return jnp.maximum(o, 0.0)


def _dense_plain(acc_ref, cnt_ref, h_ref, wl_ref, bl_ref, wr_ref, o_ref):
    o_ref[...] = _sage_block(acc_ref, cnt_ref, h_ref, wl_ref, bl_ref, wr_ref)


def _dense_fc1(acc_ref, cnt_ref, h_ref, wl_ref, bl_ref, wr_ref,
               f1w_ref, f1b_ref, o_ref):
    t = _sage_block(acc_ref, cnt_ref, h_ref, wl_ref, bl_ref, wr_ref)
    dn = (((1,), (1,)), ((), ()))
    u = lax.dot_general(t, f1w_ref[...], dn,
                        preferred_element_type=jnp.float32) + f1b_ref[...]
    o_ref[...] = jnp.maximum(u, 0.0)


def _dense_final(acc_ref, cnt_ref, h_ref, wl_ref, bl_ref, wr_ref,
                 f2w_ref, f2b_ref, o_ref):
    t = _sage_block(acc_ref, cnt_ref, h_ref, wl_ref, bl_ref, wr_ref)
    dn = (((1,), (1,)), ((), ()))
    u = lax.dot_general(t[:, :8], f2w_ref[...], dn,
                        preferred_element_type=jnp.float32) + f2b_ref[...]
    u = jnp.maximum(u, 0.0)
    m = jnp.max(u, axis=1, keepdims=True)
    e = jnp.exp(u - m)
    o_ref[...] = e / jnp.sum(e, axis=1, keepdims=True)


def _dense_call(body, acc, cnt_t, h, weights, out_cols):
    nblk = _NPAD // _R
    wspecs = [pl.BlockSpec(w.shape, lambda i, nd=w.ndim: (0,) * nd)
              for w in weights]
    return pl.pallas_call(
        body,
        grid=(nblk,),
        in_specs=[
            pl.BlockSpec((2, _R, _D), lambda i: (0, i, 0)),
            pl.BlockSpec((_R, 2), lambda i: (i, 0)),
            pl.BlockSpec((_R, _D), lambda i: (i, 0)),
        ] + wspecs,
        out_specs=pl.BlockSpec((_R, out_cols), lambda i: (i, 0)),
        out_shape=jax.ShapeDtypeStruct((_NPAD, out_cols), jnp.float32),
    )(acc, cnt_t, h, *weights)


def kernel(x, edge_index, Wl10, Wr10, Wl11, Wr11, Wl20, Wr20, Wl21, Wr21,
           bl10, bl11, bl20, bl21, fc1W, fc1b, fc2W, fc2b):
    src = edge_index[0]
    dst = edge_index[1]
    pad = _EPAD - _E
    src2d = jnp.concatenate(
        [src, jnp.zeros((pad,), jnp.int32)]).reshape(_ER, 128)
    dst2d = jnp.concatenate(
        [dst, jnp.full((pad,), _N, jnp.int32)]).reshape(_ER, 128)
    h0 = jnp.concatenate(
        [x, jnp.zeros((_NPAD - _N, _D), jnp.float32)], axis=0)
    bl10r, bl11r, bl20r, bl21r = (b.reshape(1, _D)
                                  for b in (bl10, bl11, bl20, bl21))
    f1br = fc1b.reshape(1, _D)
    f2br = fc2b.reshape(1, 8)

    acc, cnt2 = _sc_pass_cnt(h0, src2d, dst2d)
    cnt_t = jnp.transpose(cnt2, (1, 0))
    h1 = _dense_call(_dense_plain, acc, cnt_t, h0,
                     (Wl10, bl10r, Wr10), _D)
    acc = _sc_pass_acc(h1, src2d, dst2d)
    h2 = _dense_call(_dense_fc1, acc, cnt_t, h1,
                     (Wl11, bl11r, Wr11, fc1W, f1br), _D)
    acc = _sc_pass_acc(h2, src2d, dst2d)
    h3 = _dense_call(_dense_plain, acc, cnt_t, h2,
                     (Wl20, bl20r, Wr20), _D)
    acc = _sc_pass_acc(h3, src2d, dst2d)
    out = _dense_call(_dense_final, acc, cnt_t, h3,
                      (Wl21, bl21r, Wr21, fc2W, f2br), 8)
    return out[:_N]


# speedup vs baseline: 25.8882x; 25.8882x over previous
import jax
import jax.numpy as jnp
from jax import lax
from jax.experimental import pallas as pl
from jax.experimental.pallas import tpu as pltpu
from jax.experimental.pallas import tpu_sc as plsc

_N = 100000
_D = 16
_E = 3200000
_NPAD = 102400
_RT = _NPAD // 16
_ZC = 800
_NW = 32
_EPAD = 3211264
_ER = _EPAD // 128
_R = 2048


def _sc_pass(with_cnt):
    gk = 4 if with_cnt else 8
    ng = _EPAD // (_NW * gk * 128)
    group = gk * 128

    mesh = plsc.VectorSubcoreMesh(core_axis_name="c", subcore_axis_name="s")
    out_type = [jax.ShapeDtypeStruct((2, _NPAD, _D), jnp.float32)]
    scratch = [
        pltpu.VMEM((gk, 128), jnp.int32),
        pltpu.VMEM((gk, 128), jnp.int32),
        pltpu.VMEM((group, _D), jnp.float32),
        pltpu.VMEM_SHARED((_NPAD, _D), jnp.float32),
        pltpu.SemaphoreType.DMA,
    ]
    if with_cnt:
        out_type.append(jax.ShapeDtypeStruct((2, _NPAD), jnp.float32))
        scratch += [
            pltpu.VMEM((128,), jnp.float32),
            pltpu.VMEM((_ZC,), jnp.float32),
            pltpu.VMEM_SHARED((_NPAD,), jnp.float32),
        ]

    def body(h_hbm, src_hbm, dst_hbm, out_hbm, *rest):
        if with_cnt:
            (cnt_hbm, srcb, dstb, rows, acc, sem,
             ones, zcnt, cntacc) = rest
        else:
            srcb, dstb, rows, acc, sem = rest
        cid = lax.axis_index("c")
        sid = lax.axis_index("s")
        w = cid * 16 + sid

        zr = (group * 5) // 8
        def _z2(i, carry):
            rows[i, :] = jnp.zeros((_D,), jnp.float32)
            return carry
        lax.fori_loop(0, zr, _z2, 0)
        for k in range(_RT // zr):
            pltpu.sync_copy(rows.at[pl.ds(0, zr)],
                            acc.at[pl.ds(sid * _RT + k * zr, zr)])
        if with_cnt:
            def _z1(i, carry):
                zcnt[pl.ds(i * 16, 16)] = jnp.zeros((16,), jnp.float32)
                return carry
            lax.fori_loop(0, _ZC // 16, _z1, 0)
            for k in range(_RT // _ZC):
                pltpu.sync_copy(zcnt, cntacc.at[pl.ds(sid * _RT + k * _ZC,
                                                      _ZC)])
            def _o1(i, carry):
                ones[pl.ds(i * 16, 16)] = jnp.full((16,), 1.0, jnp.float32)
                return carry
            lax.fori_loop(0, 8, _o1, 0)
        plsc.subcore_barrier()

        rowbase = w * ng * gk

        def _group(g, carry):
            roff = rowbase + g * gk
            pltpu.sync_copy(src_hbm.at[pl.ds(roff, gk)], srcb)
            pltpu.sync_copy(dst_hbm.at[pl.ds(roff, gk)], dstb)
            cps = [
                pltpu.async_copy(h_hbm.at[srcb.at[j]],
                                 rows.at[pl.ds(j * 128, 128)], sem)
                for j in range(gk)
            ]
            for cp in cps:
                cp.wait()
            for j in range(gk):
                pltpu.sync_copy(rows.at[pl.ds(j * 128, 128)],
                                acc.at[dstb.at[j]], add=True)
                if with_cnt:
                    pltpu.sync_copy(ones, cntacc.at[dstb.at[j]], add=True)
            return carry
        lax.fori_loop(0, ng, _group, 0)
        plsc.subcore_barrier()

        pltpu.sync_copy(acc.at[pl.ds(sid * _RT, _RT)],
                        out_hbm.at[cid, pl.ds(sid * _RT, _RT)])
        if with_cnt:
            pltpu.sync_copy(cntacc.at[pl.ds(sid * _RT, _RT)],
                            cnt_hbm.at[cid, pl.ds(sid * _RT, _RT)])

    return pl.kernel(
        body, mesh=mesh, out_type=out_type, scratch_types=scratch,
        compiler_params=pltpu.CompilerParams(use_tc_tiling_on_sc=False))


_sc_pass_cnt = _sc_pass(True)
_sc_pass_acc = _sc_pass(False)


def _sage_block(acc_ref, cnt_ref, h_ref, wl_ref, bl_ref, wr_ref):
    a = acc_ref[0] + acc_ref[1]
    c = cnt_ref[:, 0:1] + cnt_ref[:, 1:2]
    mean = a / jnp.maximum(c, 1.0)
    dn = (((1,), (1,)), ((), ()))
    o = (lax.dot_general(mean, wl_ref[...], dn,
                         preferred_element_type=jnp.float32)
         + bl_ref[...]
         + lax.dot_general(h_ref[...], wr_ref[...], dn,
                           preferred_element_type=jnp.float32))
    return jnp.maximum(o, 0.0)


def _dense_plain(acc_ref, cnt_ref, h_ref, wl_ref, bl_ref, wr_ref, o_ref):
    o_ref[...] = _sage_block(acc_ref, cnt_ref, h_ref, wl_ref, bl_ref, wr_ref)


def _dense_fc1(acc_ref, cnt_ref, h_ref, wl_ref, bl_ref, wr_ref,
               f1w_ref, f1b_ref, o_ref):
    t = _sage_block(acc_ref, cnt_ref, h_ref, wl_ref, bl_ref, wr_ref)
    dn = (((1,), (1,)), ((), ()))
    u = lax.dot_general(t, f1w_ref[...], dn,
                        preferred_element_type=jnp.float32) + f1b_ref[...]
    o_ref[...] = jnp.maximum(u, 0.0)


def _dense_final(acc_ref, cnt_ref, h_ref, wl_ref, bl_ref, wr_ref,
                 f2w_ref, f2b_ref, o_ref):
    t = _sage_block(acc_ref, cnt_ref, h_ref, wl_ref, bl_ref, wr_ref)
    dn = (((1,), (1,)), ((), ()))
    u = lax.dot_general(t[:, :8], f2w_ref[...], dn,
                        preferred_element_type=jnp.float32) + f2b_ref[...]
    u = jnp.maximum(u, 0.0)
    m = jnp.max(u, axis=1, keepdims=True)
    e = jnp.exp(u - m)
    o_ref[...] = e / jnp.sum(e, axis=1, keepdims=True)


def _dense_call(body, acc, cnt_t, h, weights, out_cols):
    nblk = _NPAD // _R
    wspecs = [pl.BlockSpec(w.shape, lambda i, nd=w.ndim: (0,) * nd)
              for w in weights]
    return pl.pallas_call(
        body,
        grid=(nblk,),
        in_specs=[
            pl.BlockSpec((2, _R, _D), lambda i: (0, i, 0)),
            pl.BlockSpec((_R, 2), lambda i: (i, 0)),
            pl.BlockSpec((_R, _D), lambda i: (i, 0)),
        ] + wspecs,
        out_specs=pl.BlockSpec((_R, out_cols), lambda i: (i, 0)),
        out_shape=jax.ShapeDtypeStruct((_NPAD, out_cols), jnp.float32),
    )(acc, cnt_t, h, *weights)


def kernel(x, edge_index, Wl10, Wr10, Wl11, Wr11, Wl20, Wr20, Wl21, Wr21,
           bl10, bl11, bl20, bl21, fc1W, fc1b, fc2W, fc2b):
    src = edge_index[0]
    dst = edge_index[1]
    pad = _EPAD - _E
    src2d = jnp.concatenate(
        [src, jnp.zeros((pad,), jnp.int32)]).reshape(_ER, 128)
    dst2d = jnp.concatenate(
        [dst, jnp.full((pad,), _N, jnp.int32)]).reshape(_ER, 128)
    h0 = jnp.concatenate(
        [x, jnp.zeros((_NPAD - _N, _D), jnp.float32)], axis=0)
    bl10r, bl11r, bl20r, bl21r = (b.reshape(1, _D)
                                  for b in (bl10, bl11, bl20, bl21))
    f1br = fc1b.reshape(1, _D)
    f2br = fc2b.reshape(1, 8)

    acc, cnt2 = _sc_pass_cnt(h0, src2d, dst2d)
    cnt_t = jnp.transpose(cnt2, (1, 0))
    h1 = _dense_call(_dense_plain, acc, cnt_t, h0,
                     (Wl10, bl10r, Wr10), _D)
    (acc,) = _sc_pass_acc(h1, src2d, dst2d)
    h2 = _dense_call(_dense_fc1, acc, cnt_t, h1,
                     (Wl11, bl11r, Wr11, fc1W, f1br), _D)
    (acc,) = _sc_pass_acc(h2, src2d, dst2d)
    h3 = _dense_call(_dense_plain, acc, cnt_t, h2,
                     (Wl20, bl20r, Wr20), _D)
    (acc,) = _sc_pass_acc(h3, src2d, dst2d)
    out = _dense_call(_dense_final, acc, cnt_t, h3,
                      (Wl21, bl21r, Wr21, fc2W, f2br), 8)
    return out[:_N]


# speedup vs baseline: 66.8683x; 2.5830x over previous
import jax
import jax.numpy as jnp
from jax import lax
from jax.experimental import pallas as pl
from jax.experimental.pallas import tpu as pltpu
from jax.experimental.pallas import tpu_sc as plsc

_N = 100000
_D = 16
_E = 3200000
_NPAD = 102400
_M = _NPAD // 8
_RT = _NPAD // 16
_ZC = 800
_GK = 4
_NG0 = 196
_NG1 = 194
_EROWS = _E // 128
_RR = 1280


def _sc_pass(with_cnt):
    gk = _GK

    mesh = plsc.VectorSubcoreMesh(core_axis_name="c", subcore_axis_name="s")
    out_type = [jax.ShapeDtypeStruct((2, _NPAD, _D), jnp.float32)]
    scratch = [
        pltpu.VMEM((gk, 128), jnp.int32),
        pltpu.VMEM((gk, 128), jnp.int32),
        pltpu.VMEM((gk, 128), jnp.int32),
        pltpu.VMEM((gk, 128), jnp.int32),
        pltpu.VMEM((gk * 128, _D), jnp.float32),
        pltpu.VMEM((gk * 128, _D), jnp.float32),
        pltpu.VMEM_SHARED((_NPAD, _D), jnp.float32),
        pltpu.SemaphoreType.DMA,
        pltpu.SemaphoreType.DMA,
        pltpu.SemaphoreType.DMA,
        pltpu.SemaphoreType.DMA,
        pltpu.SemaphoreType.DMA,
    ]
    if with_cnt:
        out_type.append(jax.ShapeDtypeStruct((2, _NPAD), jnp.float32))
        scratch += [
            pltpu.VMEM((128,), jnp.float32),
            pltpu.VMEM((_ZC,), jnp.float32),
            pltpu.VMEM_SHARED((_NPAD,), jnp.float32),
        ]

    def body(h_hbm, e_hbm, out_hbm, *rest):
        if with_cnt:
            (cnt_hbm, sb0, db0, sb1, db1, r0, r1, acc, gsem0, gsem1,
             ssem, issem, idsem, ones, zcnt, cntacc) = rest
        else:
            (sb0, db0, sb1, db1, r0, r1, acc, gsem0, gsem1,
             ssem, issem, idsem) = rest
        srcb, dstb, rows = (sb0, sb1), (db0, db1), (r0, r1)
        gsems = (gsem0, gsem1)
        cid = lax.axis_index("c")
        sid = lax.axis_index("s")
        w = cid * 16 + sid

        zr = 320
        def _z2(i, carry):
            r0[i, :] = jnp.zeros((_D,), jnp.float32)
            return carry
        lax.fori_loop(0, zr, _z2, 0)
        for k in range(_RT // zr):
            pltpu.sync_copy(r0.at[pl.ds(0, zr)],
                            acc.at[pl.ds(sid * _RT + k * zr, zr)])
        if with_cnt:
            def _z1(i, carry):
                zcnt[pl.ds(i * 16, 16)] = jnp.zeros((16,), jnp.float32)
                return carry
            lax.fori_loop(0, _ZC // 16, _z1, 0)
            for k in range(_RT // _ZC):
                pltpu.sync_copy(zcnt, cntacc.at[pl.ds(sid * _RT + k * _ZC,
                                                      _ZC)])
            def _o1(i, carry):
                ones[pl.ds(i * 16, 16)] = jnp.full((16,), 1.0, jnp.float32)
                return carry
            lax.fori_loop(0, 8, _o1, 0)
        plsc.subcore_barrier()

        ng = jnp.where(w < 21, _NG0, _NG1)
        gbase = jnp.where(w < 21, _NG0 * w,
                          _NG0 * 21 + _NG1 * (w - 21))
        rowbase = gbase * gk

        def _roff(g):
            r = rowbase + g * gk
            return jnp.where(r >= _EROWS, r - _EROWS, r)

        def _fire_is(g, b):
            pltpu.async_copy(e_hbm.at[0, pl.ds(_roff(g), gk)],
                             srcb[b], issem)

        def _drain_is(b):
            pltpu.make_async_copy(e_hbm.at[0, pl.ds(0, gk)], srcb[b],
                                  issem).wait()

        def _fire_id(g, b):
            pltpu.async_copy(e_hbm.at[1, pl.ds(_roff(g), gk)],
                             dstb[b], idsem)

        def _drain_id(b):
            pltpu.make_async_copy(e_hbm.at[1, pl.ds(0, gk)], dstb[b],
                                  idsem).wait()

        def _fire_g(b):
            for j in range(gk):
                pltpu.async_copy(h_hbm.at[srcb[b].at[j]],
                                 rows[b].at[pl.ds(j * 128, 128)], gsems[b])

        def _drain_g(b):
            for j in range(gk):
                pltpu.make_async_copy(h_hbm.at[srcb[b].at[j]],
                                      rows[b].at[pl.ds(j * 128, 128)],
                                      gsems[b]).wait()

        def _fire_s(b):
            for j in range(gk):
                pltpu.async_copy(rows[b].at[pl.ds(j * 128, 128)],
                                 acc.at[dstb[b].at[j]], ssem, add=True)
                if with_cnt:
                    pltpu.async_copy(ones, cntacc.at[dstb[b].at[j]],
                                     ssem, add=True)

        def _drain_s(b):
            for j in range(gk):
                pltpu.make_async_copy(rows[b].at[pl.ds(j * 128, 128)],
                                      acc.at[dstb[b].at[j]], ssem).wait()
                if with_cnt:
                    pltpu.make_async_copy(ones, cntacc.at[dstb[b].at[j]],
                                          ssem).wait()

        def _section(g, b, first):
            if not first:
                _drain_s(1 - b)
            _fire_id(g + 1, 1 - b)
            _drain_is(1 - b)
            _fire_g(1 - b)
            _drain_g(b)
            _drain_id(b)
            _fire_s(b)
            _fire_is(g + 2, b)

        _fire_is(0, 0)
        _fire_id(0, 0)
        _drain_is(0)
        _fire_g(0)
        _fire_is(1, 1)
        _section(0, 0, True)
        _section(1, 1, False)

        def _pair(p, carry):
            g = 2 * p + 2
            _section(g, 0, False)
            _section(g + 1, 1, False)
            return carry
        lax.fori_loop(0, (ng - 2) // 2, _pair, 0)

        _drain_s(1)
        _drain_g(0)
        _drain_is(1)
        _drain_id(0)
        plsc.subcore_barrier()

        pltpu.sync_copy(acc.at[pl.ds(sid * _RT, _RT)],
                        out_hbm.at[cid, pl.ds(sid * _RT, _RT)])
        if with_cnt:
            pltpu.sync_copy(cntacc.at[pl.ds(sid * _RT, _RT)],
                            cnt_hbm.at[cid, pl.ds(sid * _RT, _RT)])

    return pl.kernel(
        body, mesh=mesh, out_type=out_type, scratch_types=scratch,
        compiler_params=pltpu.CompilerParams(use_tc_tiling_on_sc=False))


_sc_pass_cnt = _sc_pass(True)
_sc_pass_acc = _sc_pass(False)


_DN = (((1,), (0,)), ((), ()))


def _mm(x, w_ref):
    return lax.dot_general(x, w_ref[...], _DN,
                           precision=lax.Precision.HIGHEST,
                           preferred_element_type=jnp.float32)


def _psage(acc_ref, cnt_ref, h_ref, wl_ref, bl_ref, wr_ref):
    a = acc_ref[0] + acc_ref[1]
    c = cnt_ref[0] + cnt_ref[1]
    mean = a / jnp.maximum(c, 1.0)
    o = _mm(mean, wl_ref) + bl_ref[...] + _mm(h_ref[...], wr_ref)
    return jnp.maximum(o, 0.0)


def _pdense_plain(acc_ref, cnt_ref, h_ref, wl_ref, bl_ref, wr_ref, o_ref):
    o_ref[...] = _psage(acc_ref, cnt_ref, h_ref, wl_ref, bl_ref, wr_ref)


def _pdense_fc1(acc_ref, cnt_ref, h_ref, wl_ref, bl_ref, wr_ref,
                f1w_ref, f1b_ref, o_ref):
    t = _psage(acc_ref, cnt_ref, h_ref, wl_ref, bl_ref, wr_ref)
    u = _mm(t, f1w_ref) + f1b_ref[...]
    o_ref[...] = jnp.maximum(u, 0.0)


def _pdense_final(acc_ref, cnt_ref, h_ref, wl_ref, bl_ref, wr_ref,
                  f2w_ref, f2b_ref, sum_ref, o_ref):
    t = _psage(acc_ref, cnt_ref, h_ref, wl_ref, bl_ref, wr_ref)
    u = _mm(t, f2w_ref) + f2b_ref[...]
    u = jnp.maximum(u, 0.0)
    lane = lax.broadcasted_iota(jnp.int32, u.shape, 1)
    v = jnp.where((lane % 16) >= 8, -1e30, u)
    e = jnp.exp(v)
    s = _mm(e, sum_ref)
    o_ref[...] = e / s


def _pdense_call(body, acc_p, cnt_p, h_p, weights):
    wspecs = [pl.BlockSpec(w.shape, lambda i, nd=w.ndim: (0,) * nd)
              for w in weights]
    return pl.pallas_call(
        body,
        grid=(_M // _RR,),
        in_specs=[
            pl.BlockSpec((2, _RR, 128), lambda i: (0, i, 0)),
            pl.BlockSpec((2, _RR, 128), lambda i: (0, i, 0)),
            pl.BlockSpec((_RR, 128), lambda i: (i, 0)),
        ] + wspecs,
        out_specs=pl.BlockSpec((_RR, 128), lambda i: (i, 0)),
        out_shape=jax.ShapeDtypeStruct((_M, 128), jnp.float32),
    )(acc_p, cnt_p, h_p, *weights)


def kernel(x, edge_index, Wl10, Wr10, Wl11, Wr11, Wl20, Wr20, Wl21, Wr21,
           bl10, bl11, bl20, bl21, fc1W, fc1b, fc2W, fc2b):
    e3 = edge_index.reshape(2, _EROWS, 128)
    h0 = jnp.concatenate(
        [x, jnp.zeros((_NPAD - _N, _D), jnp.float32)], axis=0)

    eye8 = jnp.eye(8, dtype=jnp.float32)

    def _bd(wt):
        return jnp.kron(eye8, wt)

    def _brep(b):
        return jnp.tile(b, 8).reshape(1, 128)

    wl1, wr1 = _bd(Wl10.T), _bd(Wr10.T)
    wl2, wr2 = _bd(Wl11.T), _bd(Wr11.T)
    wl3, wr3 = _bd(Wl20.T), _bd(Wr20.T)
    wl4, wr4 = _bd(Wl21.T), _bd(Wr21.T)
    f1w = _bd(fc1W.T)
    f2w = _bd(jnp.concatenate(
        [jnp.concatenate([fc2W.T, jnp.zeros((8, 8), jnp.float32)], 1),
         jnp.zeros((8, 16), jnp.float32)], 0))
    f2b = _brep(jnp.concatenate([fc2b, jnp.zeros((8,), jnp.float32)]))
    smat = _bd(jnp.ones((16, 16), jnp.float32))

    acc, cnt2 = _sc_pass_cnt(h0, e3)
    acc_p = acc.reshape(2, _M, 128)
    cnt_p = jnp.repeat(cnt2.reshape(2, _M, 8), 16, axis=2)
    h1 = _pdense_call(_pdense_plain, acc_p, cnt_p, h0.reshape(_M, 128),
                      (wl1, _brep(bl10), wr1))
    (acc,) = _sc_pass_acc(h1.reshape(_NPAD, _D), e3)
    h2 = _pdense_call(_pdense_fc1, acc.reshape(2, _M, 128), cnt_p, h1,
                      (wl2, _brep(bl11), wr2, f1w, _brep(fc1b)))
    (acc,) = _sc_pass_acc(h2.reshape(_NPAD, _D), e3)
    h3 = _pdense_call(_pdense_plain, acc.reshape(2, _M, 128), cnt_p, h2,
                      (wl3, _brep(bl20), wr3))
    (acc,) = _sc_pass_acc(h3.reshape(_NPAD, _D), e3)
    out = _pdense_call(_pdense_final, acc.reshape(2, _M, 128), cnt_p, h3,
                       (wl4, _brep(bl21), wr4, f2w, f2b, smat))
    return out.reshape(_NPAD, _D)[:_N, :8]


# speedup vs baseline: 72.4747x; 1.0838x over previous
import jax
import jax.numpy as jnp
from jax import lax
from jax.experimental import pallas as pl
from jax.experimental.pallas import tpu as pltpu
from jax.experimental.pallas import tpu_sc as plsc

_N = 100000
_D = 16
_E = 3200000
_NPAD = 102400
_M = _NPAD // 8
_RT = _NPAD // 16
_ZC = 800
_EROWS = _E // 128
_RR = 1280


def _sc_pass(with_cnt, gk):
    total_pairs = _EROWS // gk // 2
    base = total_pairs // 32
    extra = total_pairs % 32
    ng_big, ng_small = 2 * (base + 1), 2 * base

    mesh = plsc.VectorSubcoreMesh(core_axis_name="c", subcore_axis_name="s")
    out_type = [jax.ShapeDtypeStruct((2, _NPAD, _D), jnp.float32)]
    scratch = [
        pltpu.VMEM((gk, 128), jnp.int32),
        pltpu.VMEM((gk, 128), jnp.int32),
        pltpu.VMEM((gk, 128), jnp.int32),
        pltpu.VMEM((gk, 128), jnp.int32),
        pltpu.VMEM((gk * 128, _D), jnp.float32),
        pltpu.VMEM((gk * 128, _D), jnp.float32),
        pltpu.VMEM_SHARED((_NPAD, _D), jnp.float32),
        pltpu.SemaphoreType.DMA,
        pltpu.SemaphoreType.DMA,
        pltpu.SemaphoreType.DMA,
        pltpu.SemaphoreType.DMA,
        pltpu.SemaphoreType.DMA,
    ]
    if with_cnt:
        out_type.append(jax.ShapeDtypeStruct((2, _NPAD), jnp.float32))
        scratch += [
            pltpu.VMEM((128,), jnp.float32),
            pltpu.VMEM((_ZC,), jnp.float32),
            pltpu.VMEM_SHARED((_NPAD,), jnp.float32),
        ]

    def body(h_hbm, e_hbm, out_hbm, *rest):
        if with_cnt:
            (cnt_hbm, sb0, db0, sb1, db1, r0, r1, acc, gsem0, gsem1,
             ssem, issem, idsem, ones, zcnt, cntacc) = rest
        else:
            (sb0, db0, sb1, db1, r0, r1, acc, gsem0, gsem1,
             ssem, issem, idsem) = rest
        srcb, dstb, rows = (sb0, sb1), (db0, db1), (r0, r1)
        gsems = (gsem0, gsem1)
        cid = lax.axis_index("c")
        sid = lax.axis_index("s")
        w = cid * 16 + sid

        zr = 320 if gk == 4 else 640
        def _z2(i, carry):
            r0[i, :] = jnp.zeros((_D,), jnp.float32)
            return carry
        lax.fori_loop(0, zr, _z2, 0)
        for k in range(_RT // zr):
            pltpu.sync_copy(r0.at[pl.ds(0, zr)],
                            acc.at[pl.ds(sid * _RT + k * zr, zr)])
        if with_cnt:
            def _z1(i, carry):
                zcnt[pl.ds(i * 16, 16)] = jnp.zeros((16,), jnp.float32)
                return carry
            lax.fori_loop(0, _ZC // 16, _z1, 0)
            for k in range(_RT // _ZC):
                pltpu.sync_copy(zcnt, cntacc.at[pl.ds(sid * _RT + k * _ZC,
                                                      _ZC)])
            def _o1(i, carry):
                ones[pl.ds(i * 16, 16)] = jnp.full((16,), 1.0, jnp.float32)
                return carry
            lax.fori_loop(0, 8, _o1, 0)
        plsc.subcore_barrier()

        ng = jnp.where(w < extra, ng_big, ng_small)
        gbase = jnp.where(w < extra, ng_big * w,
                          ng_big * extra + ng_small * (w - extra))
        rowbase = gbase * gk

        def _roff(g):
            r = rowbase + g * gk
            return jnp.where(r >= _EROWS, r - _EROWS, r)

        def _fire_is(g, b):
            pltpu.async_copy(e_hbm.at[0, pl.ds(_roff(g), gk)],
                             srcb[b], issem)

        def _drain_is(b):
            pltpu.make_async_copy(e_hbm.at[0, pl.ds(0, gk)], srcb[b],
                                  issem).wait()

        def _fire_id(g, b):
            pltpu.async_copy(e_hbm.at[1, pl.ds(_roff(g), gk)],
                             dstb[b], idsem)

        def _drain_id(b):
            pltpu.make_async_copy(e_hbm.at[1, pl.ds(0, gk)], dstb[b],
                                  idsem).wait()

        def _fire_g(b):
            for j in range(gk):
                pltpu.async_copy(h_hbm.at[srcb[b].at[j]],
                                 rows[b].at[pl.ds(j * 128, 128)], gsems[b])

        def _drain_g(b):
            for j in range(gk):
                pltpu.make_async_copy(h_hbm.at[srcb[b].at[j]],
                                      rows[b].at[pl.ds(j * 128, 128)],
                                      gsems[b]).wait()

        def _fire_s(b):
            for j in range(gk):
                pltpu.async_copy(rows[b].at[pl.ds(j * 128, 128)],
                                 acc.at[dstb[b].at[j]], ssem, add=True)
                if with_cnt:
                    pltpu.async_copy(ones, cntacc.at[dstb[b].at[j]],
                                     ssem, add=True)

        def _drain_s(b):
            for j in range(gk):
                pltpu.make_async_copy(rows[b].at[pl.ds(j * 128, 128)],
                                      acc.at[dstb[b].at[j]], ssem).wait()
                if with_cnt:
                    pltpu.make_async_copy(ones, cntacc.at[dstb[b].at[j]],
                                          ssem).wait()

        def _section(g, b, first):
            if not first:
                _drain_s(1 - b)
            _fire_id(g + 1, 1 - b)
            _drain_is(1 - b)
            _fire_g(1 - b)
            _drain_g(b)
            _drain_id(b)
            _fire_s(b)
            _fire_is(g + 2, b)

        _fire_is(0, 0)
        _fire_id(0, 0)
        _drain_is(0)
        _fire_g(0)
        _fire_is(1, 1)
        _section(0, 0, True)
        _section(1, 1, False)

        def _pair(p, carry):
            g = 2 * p + 2
            _section(g, 0, False)
            _section(g + 1, 1, False)
            return carry
        lax.fori_loop(0, (ng - 2) // 2, _pair, 0)

        _drain_s(1)
        _drain_g(0)
        _drain_is(1)
        _drain_id(0)
        plsc.subcore_barrier()

        pltpu.sync_copy(acc.at[pl.ds(sid * _RT, _RT)],
                        out_hbm.at[cid, pl.ds(sid * _RT, _RT)])
        if with_cnt:
            pltpu.sync_copy(cntacc.at[pl.ds(sid * _RT, _RT)],
                            cnt_hbm.at[cid, pl.ds(sid * _RT, _RT)])

    return pl.kernel(
        body, mesh=mesh, out_type=out_type, scratch_types=scratch,
        compiler_params=pltpu.CompilerParams(use_tc_tiling_on_sc=False))


_sc_pass_cnt = _sc_pass(True, 4)
_sc_pass_acc = _sc_pass(False, 5)


_DN = (((1,), (0,)), ((), ()))


def _mm(x, w_ref):
    return lax.dot_general(x, w_ref[...], _DN,
                           precision=lax.Precision.HIGHEST,
                           preferred_element_type=jnp.float32)


def _psage(acc_ref, cnt_ref, h_ref, wl_ref, bl_ref, wr_ref):
    a = acc_ref[0] + acc_ref[1]
    c = cnt_ref[0] + cnt_ref[1]
    mean = a / jnp.maximum(c, 1.0)
    o = _mm(mean, wl_ref) + bl_ref[...] + _mm(h_ref[...], wr_ref)
    return jnp.maximum(o, 0.0)


def _pdense_plain(acc_ref, cnt_ref, h_ref, wl_ref, bl_ref, wr_ref, o_ref):
    o_ref[...] = _psage(acc_ref, cnt_ref, h_ref, wl_ref, bl_ref, wr_ref)


def _pdense_fc1(acc_ref, cnt_ref, h_ref, wl_ref, bl_ref, wr_ref,
                f1w_ref, f1b_ref, o_ref):
    t = _psage(acc_ref, cnt_ref, h_ref, wl_ref, bl_ref, wr_ref)
    u = _mm(t, f1w_ref) + f1b_ref[...]
    o_ref[...] = jnp.maximum(u, 0.0)


def _pdense_final(acc_ref, cnt_ref, h_ref, wl_ref, bl_ref, wr_ref,
                  f2w_ref, f2b_ref, c_ref, sc_ref, o_ref):
    t = _psage(acc_ref, cnt_ref, h_ref, wl_ref, bl_ref, wr_ref)
    u = _mm(t, f2w_ref) + f2b_ref[...]
    u = jnp.maximum(u, 0.0)
    lane = lax.broadcasted_iota(jnp.int32, u.shape, 1)
    v = jnp.where((lane % 16) >= 8, -1e30, u)
    e = jnp.exp(v)
    o_ref[...] = _mm(e, c_ref) / _mm(e, sc_ref)


def _pdense_call(body, acc_p, cnt_p, h_p, weights, out_block=(_RR, 128),
                 out_shape=(_M, 128)):
    wspecs = [pl.BlockSpec(w.shape, lambda i, nd=w.ndim: (0,) * nd)
              for w in weights]
    return pl.pallas_call(
        body,
        grid=(_M // _RR,),
        in_specs=[
            pl.BlockSpec((2, _RR, 128), lambda i: (0, i, 0)),
            pl.BlockSpec((2, _RR, 128), lambda i: (0, i, 0)),
            pl.BlockSpec((_RR, 128), lambda i: (i, 0)),
        ] + wspecs,
        out_specs=pl.BlockSpec(out_block, lambda i: (i, 0)),
        out_shape=jax.ShapeDtypeStruct(out_shape, jnp.float32),
    )(acc_p, cnt_p, h_p, *weights)


def kernel(x, edge_index, Wl10, Wr10, Wl11, Wr11, Wl20, Wr20, Wl21, Wr21,
           bl10, bl11, bl20, bl21, fc1W, fc1b, fc2W, fc2b):
    e3 = edge_index.reshape(2, _EROWS, 128)
    h0 = jnp.concatenate(
        [x, jnp.zeros((_NPAD - _N, _D), jnp.float32)], axis=0)

    eye8 = jnp.eye(8, dtype=jnp.float32)

    def _bd(wt):
        return jnp.kron(eye8, wt)

    def _brep(b):
        return jnp.tile(b, 8).reshape(1, 128)

    wl1, wr1 = _bd(Wl10.T), _bd(Wr10.T)
    wl2, wr2 = _bd(Wl11.T), _bd(Wr11.T)
    wl3, wr3 = _bd(Wl20.T), _bd(Wr20.T)
    wl4, wr4 = _bd(Wl21.T), _bd(Wr21.T)
    f1w = _bd(fc1W.T)
    f2w = _bd(jnp.concatenate(
        [jnp.concatenate([fc2W.T, jnp.zeros((8, 8), jnp.float32)], 1),
         jnp.zeros((8, 16), jnp.float32)], 0))
    f2b = _brep(jnp.concatenate([fc2b, jnp.zeros((8,), jnp.float32)]))
    cmat = jnp.kron(eye8, jnp.concatenate(
        [jnp.eye(8, dtype=jnp.float32),
         jnp.zeros((8, 8), jnp.float32)], axis=0))
    scmat = jnp.kron(eye8, jnp.ones((16, 8), jnp.float32))

    acc, cnt2 = _sc_pass_cnt(h0, e3)
    acc_p = acc.reshape(2, _M, 128)
    cnt_p = jnp.repeat(cnt2.reshape(2, _M, 8), 16, axis=2)
    h1 = _pdense_call(_pdense_plain, acc_p, cnt_p, h0.reshape(_M, 128),
                      (wl1, _brep(bl10), wr1))
    (acc,) = _sc_pass_acc(h1.reshape(_NPAD, _D), e3)
    h2 = _pdense_call(_pdense_fc1, acc.reshape(2, _M, 128), cnt_p, h1,
                      (wl2, _brep(bl11), wr2, f1w, _brep(fc1b)))
    (acc,) = _sc_pass_acc(h2.reshape(_NPAD, _D), e3)
    h3 = _pdense_call(_pdense_plain, acc.reshape(2, _M, 128), cnt_p, h2,
                      (wl3, _brep(bl20), wr3))
    (acc,) = _sc_pass_acc(h3.reshape(_NPAD, _D), e3)
    out = _pdense_call(_pdense_final, acc.reshape(2, _M, 128), cnt_p, h3,
                       (wl4, _brep(bl21), wr4, f2w, f2b, cmat, scmat),
                       out_block=(_RR, 64), out_shape=(_M, 64))
    return out.reshape(_NPAD, 8)[:_N]
